# baseline (speedup 1.0000x reference)
import functools

import jax
import jax.numpy as jnp
from jax import lax
from jax.experimental import pallas as pl
from jax.experimental.pallas import tpu as pltpu
from jax.experimental.pallas import tpu_sc as plsc

HIST = 50
DM = 64
PADW = 128
GROUP = 128
NBUF = 10
K = 6
NW = 32
LANES = 16
TBLK = 8192
BJC = 1024


@functools.partial(jax.jit, static_argnums=(2,))
def _sc_gather(w_pad, idx_grouped, ngroups):
    b_total = NW * ngroups * GROUP
    mesh = plsc.VectorSubcoreMesh(core_axis_name="c", subcore_axis_name="s")
    nc = plsc.get_sparse_core_info().num_cores

    @functools.partial(
        pl.kernel,
        mesh=mesh,
        out_type=jax.ShapeDtypeStruct((b_total, DM), jnp.float32),
        scratch_types=[
            pltpu.VMEM((ngroups * GROUP,), jnp.int32),
            pltpu.VMEM((NBUF, GROUP), jnp.int32),
            pltpu.VMEM((NBUF, GROUP, DM), jnp.float32),
            pltpu.SemaphoreType.DMA((NBUF,)),
            pltpu.SemaphoreType.DMA((NBUF,)),
        ],
        compiler_params=pltpu.CompilerParams(use_tc_tiling_on_sc=False,
                                             needs_layout_passes=False),
    )
    def k(w_hbm, idx_hbm, out_hbm, tok_v, sidx_v, rows_v, gsem, ssem):
        wid = lax.axis_index("s") * nc + lax.axis_index("c")
        base = wid * (ngroups * GROUP)
        pltpu.sync_copy(idx_hbm.at[wid], tok_v)

        ii = lax.iota(jnp.int32, LANES)
        pat0 = lax.shift_left(jnp.bitwise_and(ii, 1), 7) + \
            lax.shift_right_logical(ii, 1)

        def gather_start(g, b):
            s0 = (g // 2) * 256 + (g % 2) * 64
            for c in range(GROUP // LANES):
                lanes = pat0 + (s0 + 8 * c)
                tok = plsc.load_gather(tok_v, [lanes])
                row = (jnp.bitwise_and(tok, ~(TBLK - 1))
                       + lax.shift_left(
                           jnp.bitwise_and(tok, TBLK // 2 - 1), 1)
                       + jnp.bitwise_and(
                           lax.shift_right_logical(tok, 12), 1))
                sidx_v[b, pl.ds(c * LANES, LANES)] = row
            pltpu.async_copy(w_hbm.at[sidx_v.at[b]], rows_v.at[b], gsem.at[b])

        def gather_wait(g, b):
            pltpu.make_async_copy(w_hbm.at[sidx_v.at[b]], rows_v.at[b],
                                  gsem.at[b]).wait()

        def scat_start(g, b):
            pltpu.async_copy(rows_v.at[b],
                             out_hbm.at[pl.ds(base + g * GROUP, GROUP)],
                             ssem.at[b])

        def scat_wait(g, b):
            pltpu.make_async_copy(rows_v.at[b],
                                  out_hbm.at[pl.ds(base + g * GROUP, GROUP)],
                                  ssem.at[b]).wait()

        for b in range(K):
            gather_start(b, b)

        def outer(t, _):
            for j in range(NBUF):
                g = t * NBUF + j
                gather_wait(g, j)
                scat_start(g, j)
                gn = g + K
                bn = (j + K) % NBUF

                @pl.when(gn < ngroups)
                def _():
                    @pl.when(gn >= NBUF)
                    def _():
                        scat_wait(gn - NBUF, bn)

                    gather_start(gn, bn)

            return 0

        lax.fori_loop(0, ngroups // NBUF, outer, 0)
        for b in range(NBUF):
            scat_wait(ngroups - NBUF + b, b)

    return k(w_pad, idx_grouped)


def _mxu_t(x):
    eye = jnp.eye(DM, dtype=jnp.float32)
    return lax.dot_general(x, eye, (((0,), (0,)), ((), ())),
                           precision=lax.Precision.HIGHEST,
                           preferred_element_type=jnp.float32)


def _tc_pad_table_body(wt_ref, o_ref):
    blk = wt_ref[...]
    half = blk.shape[1] // 2
    o_ref[...] = jnp.concatenate(
        [blk[:, :half].T, blk[:, half:].T], axis=1)


@jax.jit
def _tc_pad_table(wt):
    v = wt.shape[1]
    grid = (v + TBLK - 1) // TBLK
    return pl.pallas_call(
        _tc_pad_table_body,
        grid=(grid,),
        in_specs=[pl.BlockSpec((DM, TBLK), lambda c: (0, c))],
        out_specs=pl.BlockSpec((TBLK // 2, PADW), lambda c: (c, 0)),
        out_shape=jax.ShapeDtypeStruct((grid * TBLK // 2, PADW), jnp.float32),
    )(wt)


def _tc_transpose_body(x_ref, o_ref):
    blk = x_ref[0]
    ev = blk[:, :DM].T
    od = blk[:, DM:].T
    et = ev.reshape(8, 8, DM, GROUP).transpose(0, 2, 1, 3)
    ot = od.reshape(8, 8, DM, GROUP).transpose(0, 2, 1, 3)
    o_ref[0] = jnp.stack([et, ot], axis=2).reshape(8, GROUP, 8, GROUP)


def _tc_transpose_alias_body(x_ref, a_ref, o_ref):
    del a_ref
    _tc_transpose_body(x_ref, o_ref)


@functools.partial(jax.jit, static_argnums=(2,))
def _tc_transpose_into(x, acc, h_off):
    hs, npr = x.shape[0], x.shape[1]
    nbj = npr // DM
    return pl.pallas_call(
        _tc_transpose_alias_body,
        grid=(hs,),
        in_specs=[
            pl.BlockSpec((1, npr, PADW), lambda h: (h, 0, 0)),
            pl.BlockSpec(memory_space=pl.ANY),
        ],
        out_specs=pl.BlockSpec((1, 8, nbj, 8, GROUP),
                               lambda h: (h + h_off, 0, 0, 0, 0)),
        out_shape=jax.ShapeDtypeStruct((HIST, 8, nbj, 8, GROUP), jnp.float32),
        input_output_aliases={1: 0},
    )(x, acc)


@jax.jit
def _tc_transpose_first(x):
    hs, npr = x.shape[0], x.shape[1]
    nbj = npr // DM
    return pl.pallas_call(
        _tc_transpose_body,
        grid=(hs,),
        in_specs=[pl.BlockSpec((1, npr, PADW), lambda h: (h, 0, 0))],
        out_specs=pl.BlockSpec((1, 8, nbj, 8, GROUP),
                               lambda h: (h, 0, 0, 0, 0)),
        out_shape=jax.ShapeDtypeStruct((HIST, 8, nbj, 8, GROUP), jnp.float32),
    )(x)


@jax.jit
def _tc_transpose(x):
    hs, bsz = x.shape[0], x.shape[1]
    nbj = bsz // GROUP
    return pl.pallas_call(
        _tc_transpose_body,
        grid=(hs,),
        in_specs=[pl.BlockSpec((1, bsz, PADW), lambda h: (h, 0, 0))],
        out_specs=pl.BlockSpec((1, 8, nbj, 8, GROUP),
                               lambda h: (h, 0, 0, 0, 0)),
        out_shape=jax.ShapeDtypeStruct((hs, 8, nbj, 8, GROUP), jnp.float32),
    )(x)


def kernel(token_ids, weight):
    bsz, h = token_ids.shape
    idx_t = token_ids.astype(jnp.int32).T
    w2 = _tc_pad_table(weight.T).reshape(-1, DM)
    npr = bsz * DM // GROUP
    chunks = [10, 15, 15, 10] if h == 50 else [h]
    rows = []
    off = 0
    for hc in chunks:
        ng = hc * bsz // (NW * GROUP)
        rows.append(_sc_gather(w2, idx_t[off:off + hc].reshape(
            NW, ng * GROUP), ng))
        off += hc
    acc = _tc_transpose_first(rows[0].reshape(chunks[0], npr, PADW))
    off = chunks[0]
    for hc, r in zip(chunks[1:], rows[1:]):
        acc = _tc_transpose_into(r.reshape(hc, npr, PADW), acc, off)
        off += hc
    out5 = acc
    return out5.transpose(2, 4, 0, 1, 3).reshape(bsz, h, DM)


# speedup vs baseline: 1.0071x; 1.0071x over previous
import functools

import jax
import jax.numpy as jnp
from jax import lax
from jax.experimental import pallas as pl
from jax.experimental.pallas import tpu as pltpu
from jax.experimental.pallas import tpu_sc as plsc

HIST = 50
DM = 64
PADW = 128
GROUP = 128
NBUF = 4
K = 3
NW = 32
LANES = 16
TBLK = 8192
BJC = 1024


@functools.partial(jax.jit, static_argnums=(2,))
def _sc_gather(w_pad, idx_grouped, ngroups):
    b_total = NW * ngroups * GROUP
    mesh = plsc.VectorSubcoreMesh(core_axis_name="c", subcore_axis_name="s")
    nc = plsc.get_sparse_core_info().num_cores

    @functools.partial(
        pl.kernel,
        mesh=mesh,
        out_type=jax.ShapeDtypeStruct((b_total, DM), jnp.float32),
        scratch_types=[
            pltpu.VMEM((ngroups * GROUP,), jnp.int32),
            pltpu.VMEM((NBUF, GROUP), jnp.int32),
            pltpu.VMEM((NBUF, GROUP, DM), jnp.float32),
            pltpu.SemaphoreType.DMA((NBUF,)),
            pltpu.SemaphoreType.DMA((NBUF,)),
        ],
        compiler_params=pltpu.CompilerParams(use_tc_tiling_on_sc=False,
                                             needs_layout_passes=False),
    )
    def k(w_hbm, idx_hbm, out_hbm, tok_v, sidx_v, rows_v, gsem, ssem):
        wid = lax.axis_index("s") * nc + lax.axis_index("c")
        base = wid * (ngroups * GROUP)
        pltpu.sync_copy(idx_hbm.at[wid], tok_v)

        ii = lax.iota(jnp.int32, LANES)
        pat0 = lax.shift_left(jnp.bitwise_and(ii, 1), 7) + \
            lax.shift_right_logical(ii, 1)

        def gather_start(g, b):
            s0 = (g // 2) * 256 + (g % 2) * 64
            for c in range(GROUP // LANES):
                lanes = pat0 + (s0 + 8 * c)
                tok = plsc.load_gather(tok_v, [lanes])
                row = (jnp.bitwise_and(tok, ~(TBLK - 1))
                       + lax.shift_left(
                           jnp.bitwise_and(tok, TBLK // 2 - 1), 1)
                       + jnp.bitwise_and(
                           lax.shift_right_logical(tok, 12), 1))
                sidx_v[b, pl.ds(c * LANES, LANES)] = row
            pltpu.async_copy(w_hbm.at[sidx_v.at[b]], rows_v.at[b], gsem.at[b])

        def gather_wait(g, b):
            pltpu.make_async_copy(w_hbm.at[sidx_v.at[b]], rows_v.at[b],
                                  gsem.at[b]).wait()

        def scat_start(g, b):
            pltpu.async_copy(rows_v.at[b],
                             out_hbm.at[pl.ds(base + g * GROUP, GROUP)],
                             ssem.at[b])

        def scat_wait(g, b):
            pltpu.make_async_copy(rows_v.at[b],
                                  out_hbm.at[pl.ds(base + g * GROUP, GROUP)],
                                  ssem.at[b]).wait()

        for b in range(K):
            gather_start(b, b)

        def outer(t, _):
            for j in range(NBUF):
                g = t * NBUF + j
                gather_wait(g, j)
                scat_start(g, j)
                gn = g + K
                bn = (j + K) % NBUF

                @pl.when(gn < ngroups)
                def _():
                    @pl.when(gn >= NBUF)
                    def _():
                        scat_wait(gn - NBUF, bn)

                    gather_start(gn, bn)

            return 0

        lax.fori_loop(0, ngroups // NBUF, outer, 0)
        for b in range(NBUF):
            scat_wait(ngroups - NBUF + b, b)

    return k(w_pad, idx_grouped)


def _mxu_t(x):
    eye = jnp.eye(DM, dtype=jnp.float32)
    return lax.dot_general(x, eye, (((0,), (0,)), ((), ())),
                           precision=lax.Precision.HIGHEST,
                           preferred_element_type=jnp.float32)


def _tc_pad_table_body(wt_ref, o_ref):
    blk = wt_ref[...]
    half = blk.shape[1] // 2
    o_ref[...] = jnp.concatenate(
        [blk[:, :half].T, blk[:, half:].T], axis=1)


@jax.jit
def _tc_pad_table(wt):
    v = wt.shape[1]
    grid = (v + TBLK - 1) // TBLK
    return pl.pallas_call(
        _tc_pad_table_body,
        grid=(grid,),
        in_specs=[pl.BlockSpec((DM, TBLK), lambda c: (0, c))],
        out_specs=pl.BlockSpec((TBLK // 2, PADW), lambda c: (c, 0)),
        out_shape=jax.ShapeDtypeStruct((grid * TBLK // 2, PADW), jnp.float32),
    )(wt)


def _tc_transpose_body(x_ref, o_ref):
    blk = x_ref[0]
    ev = blk[:, :DM].T
    od = blk[:, DM:].T
    et = ev.reshape(8, 8, DM, GROUP).transpose(0, 2, 1, 3)
    ot = od.reshape(8, 8, DM, GROUP).transpose(0, 2, 1, 3)
    o_ref[0] = jnp.stack([et, ot], axis=2).reshape(8, GROUP, 8, GROUP)


def _tc_transpose_alias_body(x_ref, a_ref, o_ref):
    del a_ref
    _tc_transpose_body(x_ref, o_ref)


@functools.partial(jax.jit, static_argnums=(2,))
def _tc_transpose_into(x, acc, h_off):
    hs, npr = x.shape[0], x.shape[1]
    nbj = npr // DM
    return pl.pallas_call(
        _tc_transpose_alias_body,
        grid=(hs,),
        in_specs=[
            pl.BlockSpec((1, npr, PADW), lambda h: (h, 0, 0)),
            pl.BlockSpec(memory_space=pl.ANY),
        ],
        out_specs=pl.BlockSpec((1, 8, nbj, 8, GROUP),
                               lambda h: (h + h_off, 0, 0, 0, 0)),
        out_shape=jax.ShapeDtypeStruct((HIST, 8, nbj, 8, GROUP), jnp.float32),
        input_output_aliases={1: 0},
    )(x, acc)


@jax.jit
def _tc_transpose_first(x):
    hs, npr = x.shape[0], x.shape[1]
    nbj = npr // DM
    return pl.pallas_call(
        _tc_transpose_body,
        grid=(hs,),
        in_specs=[pl.BlockSpec((1, npr, PADW), lambda h: (h, 0, 0))],
        out_specs=pl.BlockSpec((1, 8, nbj, 8, GROUP),
                               lambda h: (h, 0, 0, 0, 0)),
        out_shape=jax.ShapeDtypeStruct((HIST, 8, nbj, 8, GROUP), jnp.float32),
    )(x)


@jax.jit
def _tc_transpose(x):
    hs, bsz = x.shape[0], x.shape[1]
    nbj = bsz // GROUP
    return pl.pallas_call(
        _tc_transpose_body,
        grid=(hs,),
        in_specs=[pl.BlockSpec((1, bsz, PADW), lambda h: (h, 0, 0))],
        out_specs=pl.BlockSpec((1, 8, nbj, 8, GROUP),
                               lambda h: (h, 0, 0, 0, 0)),
        out_shape=jax.ShapeDtypeStruct((hs, 8, nbj, 8, GROUP), jnp.float32),
    )(x)


def kernel(token_ids, weight):
    bsz, h = token_ids.shape
    idx_t = token_ids.astype(jnp.int32).T
    w2 = _tc_pad_table(weight.T).reshape(-1, DM)
    npr = bsz * DM // GROUP
    chunks = [13, 13, 12, 12] if h == 50 else [h]
    rows = []
    off = 0
    for hc in chunks:
        ng = hc * bsz // (NW * GROUP)
        rows.append(_sc_gather(w2, idx_t[off:off + hc].reshape(
            NW, ng * GROUP), ng))
        off += hc
    acc = _tc_transpose_first(rows[0].reshape(chunks[0], npr, PADW))
    off = chunks[0]
    for hc, r in zip(chunks[1:], rows[1:]):
        acc = _tc_transpose_into(r.reshape(hc, npr, PADW), acc, off)
        off += hc
    out5 = acc
    return out5.transpose(2, 4, 0, 1, 3).reshape(bsz, h, DM)
